# Initial kernel scaffold; baseline (speedup 1.0000x reference)
import functools

import jax
import jax.numpy as jnp
from jax import lax
from jax.experimental import pallas as pl
from jax.experimental.pallas import tpu as pltpu
from jax.experimental.pallas import tpu_sc as plsc

_N = 10000
_E = 320000
_DIN = 128
_DH = 64
_DOUT = 128

_NC = 2
_NS = 16
_NT = _NC * _NS
_CHUNK = 128
_NCH = 80
_PER_TILE = _CHUNK * _NCH
_EPAD = _NT * _PER_TILE
_NACC = 10240
_RPT = _NACC // _NS
_NZC = _RPT // _CHUNK
_DEGW = 16


def _make_seg_sum(with_deg):
    mesh = plsc.VectorSubcoreMesh(core_axis_name="c", subcore_axis_name="s")
    out_types = [jax.ShapeDtypeStruct((_NC * _NACC, _DH), jnp.float32)]
    scratch = [
        pltpu.VMEM((_NCH, _CHUNK), jnp.int32),
        pltpu.VMEM((_NCH, _CHUNK), jnp.int32),
        pltpu.VMEM((_CHUNK, _DH), jnp.float32),
        pltpu.VMEM((_CHUNK, _DH), jnp.float32),
        pltpu.VMEM_SHARED((_NACC, _DH), jnp.float32),
        pltpu.SemaphoreType.DMA,
    ]
    if with_deg:
        out_types.append(jax.ShapeDtypeStruct((_NC * _NACC, _DEGW), jnp.float32))
        scratch += [
            pltpu.VMEM((_CHUNK, _DEGW), jnp.float32),
            pltpu.VMEM_SHARED((_NACC, _DEGW), jnp.float32),
        ]

    @functools.partial(pl.kernel, mesh=mesh, out_type=out_types,
                       scratch_types=scratch)
    def seg(vals, srcp, dstp, *rest):
        if with_deg:
            out, outd, src_v, dst_v, rb0, rb1, acc, gsem, ones_v, accd = rest
        else:
            out, src_v, dst_v, rb0, rb1, acc, gsem = rest
        c = lax.axis_index("c")
        s = lax.axis_index("s")
        tid = c * _NS + s
        rowb = s * _RPT

        zero16 = jnp.zeros((16,), jnp.float32)

        def zrow(i, _):
            for j in range(_DH // 16):
                rb0[i, pl.ds(j * 16, 16)] = zero16
            return 0
        lax.fori_loop(0, _CHUNK, zrow, 0)

        def zcp(k, _):
            pltpu.sync_copy(rb0, acc.at[pl.ds(rowb + k * _CHUNK, _CHUNK)])
            return 0
        lax.fori_loop(0, _NZC, zcp, 0)

        if with_deg:
            def zdrow(i, _):
                ones_v[i, pl.ds(0, 16)] = zero16
                return 0
            lax.fori_loop(0, _CHUNK, zdrow, 0)

            def zdcp(k, _):
                pltpu.sync_copy(ones_v, accd.at[pl.ds(rowb + k * _CHUNK, _CHUNK)])
                return 0
            lax.fori_loop(0, _NZC, zdcp, 0)

            one16 = jnp.full((16,), 1.0, jnp.float32)

            def orow(i, _):
                ones_v[i, pl.ds(0, 16)] = one16
                return 0
            lax.fori_loop(0, _CHUNK, orow, 0)

        pltpu.sync_copy(srcp.at[tid], src_v)
        pltpu.sync_copy(dstp.at[tid], dst_v)

        plsc.subcore_barrier()

        pltpu.async_copy(vals.at[src_v.at[0]], rb0, gsem)

        def chunk(j, _):
            even = j % 2 == 0

            def do(buf, nxt):
                @pl.when(j + 1 < _NCH)
                def _():
                    pltpu.async_copy(vals.at[src_v.at[j + 1]], nxt, gsem)
                pltpu.make_async_copy(vals.at[src_v.at[j]], buf, gsem).wait()
                pltpu.sync_copy(buf, acc.at[dst_v.at[j]], add=True)
                if with_deg:
                    pltpu.sync_copy(ones_v, accd.at[dst_v.at[j]], add=True)

            @pl.when(even)
            def _():
                do(rb0, rb1)

            @pl.when(jnp.logical_not(even))
            def _():
                do(rb1, rb0)
            return 0
        lax.fori_loop(0, _NCH, chunk, 0)

        plsc.subcore_barrier()

        ob = c * _NACC + rowb

        def wcp(k, _):
            pltpu.sync_copy(acc.at[pl.ds(rowb + k * _CHUNK, _CHUNK)], rb0)
            pltpu.sync_copy(rb0, out.at[pl.ds(ob + k * _CHUNK, _CHUNK)])
            return 0
        lax.fori_loop(0, _NZC, wcp, 0)

        if with_deg:
            def wdcp(k, _):
                pltpu.sync_copy(accd.at[pl.ds(rowb + k * _CHUNK, _CHUNK)], ones_v)
                pltpu.sync_copy(ones_v, outd.at[pl.ds(ob + k * _CHUNK, _CHUNK)])
                return 0
            lax.fori_loop(0, _NZC, wdcp, 0)

    return seg


_seg_deg = _make_seg_sum(with_deg=True)
_seg = _make_seg_sum(with_deg=False)


def _tc_pre(x, W1l, W1r):
    def body(x_ref, a_ref, b_ref, y_ref, z_ref):
        xv = x_ref[...]
        y_ref[...] = jnp.dot(xv, a_ref[...], preferred_element_type=jnp.float32)
        z_ref[...] = jnp.dot(xv, b_ref[...], preferred_element_type=jnp.float32)

    return pl.pallas_call(
        body,
        out_shape=[jax.ShapeDtypeStruct((_N, _DH), jnp.float32),
                   jax.ShapeDtypeStruct((_N, _DH), jnp.float32)],
    )(x, W1l, W1r)


def _tc_mid(p1, pd, z1, b1, W2r):
    def body(p_ref, pd_ref, z1_ref, b1_ref, w_ref, h_ref, z2_ref):
        agg = p_ref[0:_N, :] + p_ref[_NACC:_NACC + _N, :]
        deg = pd_ref[0:_N, 0:1] + pd_ref[_NACC:_NACC + _N, 0:1]
        mean = agg / jnp.maximum(deg, 1.0)
        h = jnp.maximum(mean + b1_ref[...] + z1_ref[...], 0.0)
        h_ref[...] = h
        z2_ref[...] = jnp.dot(h, w_ref[...], preferred_element_type=jnp.float32)

    return pl.pallas_call(
        body,
        out_shape=[jax.ShapeDtypeStruct((_N, _DH), jnp.float32),
                   jax.ShapeDtypeStruct((_N, _DOUT), jnp.float32)],
    )(p1, pd, z1, b1, W2r)


def _tc_post(p2, pd, z2, b2, W2l):
    def body(p_ref, pd_ref, z2_ref, b2_ref, w_ref, o_ref):
        agg = p_ref[0:_N, :] + p_ref[_NACC:_NACC + _N, :]
        deg = pd_ref[0:_N, 0:1] + pd_ref[_NACC:_NACC + _N, 0:1]
        mean = agg / jnp.maximum(deg, 1.0)
        o_ref[...] = (jnp.dot(mean, w_ref[...], preferred_element_type=jnp.float32)
                      + b2_ref[...] + z2_ref[...])

    return pl.pallas_call(
        body,
        out_shape=jax.ShapeDtypeStruct((_N, _DOUT), jnp.float32),
    )(p2, pd, z2, b2, W2l)


def _first(res):
    return res[0] if isinstance(res, (list, tuple)) else res


def kernel(x, edge_index, W1l, b1, W1r, W2l, b2, W2r):
    src = edge_index[0]
    dst = edge_index[1]
    pad = _EPAD - _E
    ar = jnp.arange(pad, dtype=jnp.int32)
    srcp = jnp.concatenate([src, ar % 16]).reshape(_NT, _NCH, _CHUNK)
    dstp = jnp.concatenate([dst, _N + (ar % 16)]).reshape(_NT, _NCH, _CHUNK)

    y1, z1 = _tc_pre(x, W1l, W1r)
    p1, pd = _seg_deg(y1, srcp, dstp)
    h, z2 = _tc_mid(p1, pd, z1, b1.reshape(1, _DH), W2r)
    p2 = _first(_seg(h, srcp, dstp))
    out = _tc_post(p2, pd, z2, b2.reshape(1, _DOUT), W2l)
    return out


# speedup vs baseline: 13.2525x; 13.2525x over previous
import functools

import jax
import jax.numpy as jnp
from jax import lax
from jax.experimental import pallas as pl
from jax.experimental.pallas import tpu as pltpu
from jax.experimental.pallas import tpu_sc as plsc

_N = 10000
_E = 320000
_DIN = 128
_DH = 64
_DOUT = 128

_NC = 2
_NS = 16
_NT = _NC * _NS
_CHUNK = 128
_NCH = 80
_PER_TILE = _CHUNK * _NCH
_EPAD = _NT * _PER_TILE
_NACC = 10240
_RPT = _NACC // _NS
_NZC = _RPT // _CHUNK
_DEGW = 16


@functools.lru_cache(maxsize=None)
def _make_seg_sum(with_deg):
    mesh = plsc.VectorSubcoreMesh(core_axis_name="c", subcore_axis_name="s")
    out_types = [jax.ShapeDtypeStruct((_NC * _NACC, _DH), jnp.float32)]
    scratch = [
        pltpu.VMEM((_NCH, _CHUNK), jnp.int32),
        pltpu.VMEM((_NCH, _CHUNK), jnp.int32),
        pltpu.VMEM((_CHUNK, _DH), jnp.float32),
        pltpu.VMEM((_CHUNK, _DH), jnp.float32),
        pltpu.VMEM_SHARED((_NACC, _DH), jnp.float32),
        pltpu.SemaphoreType.DMA,
    ]
    if with_deg:
        out_types.append(jax.ShapeDtypeStruct((_NC * _NACC, _DEGW), jnp.float32))
        scratch += [
            pltpu.VMEM((_CHUNK, _DEGW), jnp.float32),
            pltpu.VMEM_SHARED((_NACC, _DEGW), jnp.float32),
        ]

    @functools.partial(
        pl.kernel, mesh=mesh, out_type=out_types, scratch_types=scratch,
        compiler_params=pltpu.CompilerParams(use_tc_tiling_on_sc=False))
    def seg(vals, srcp, dstp, *rest):
        if with_deg:
            out, outd, src_v, dst_v, rb0, rb1, acc, gsem, ones_v, accd = rest
        else:
            out, src_v, dst_v, rb0, rb1, acc, gsem = rest
        c = lax.axis_index("c")
        s = lax.axis_index("s")
        tid = c * _NS + s
        rowb = s * _RPT

        zero16 = jnp.zeros((16,), jnp.float32)

        def zrow(i, _):
            for j in range(_DH // 16):
                rb0[i, pl.ds(j * 16, 16)] = zero16
            return 0
        lax.fori_loop(0, _CHUNK, zrow, 0)

        def zcp(k, _):
            pltpu.sync_copy(rb0, acc.at[pl.ds(rowb + k * _CHUNK, _CHUNK)])
            return 0
        lax.fori_loop(0, _NZC, zcp, 0)

        if with_deg:
            def zdrow(i, _):
                ones_v[i, pl.ds(0, 16)] = zero16
                return 0
            lax.fori_loop(0, _CHUNK, zdrow, 0)

            def zdcp(k, _):
                pltpu.sync_copy(ones_v, accd.at[pl.ds(rowb + k * _CHUNK, _CHUNK)])
                return 0
            lax.fori_loop(0, _NZC, zdcp, 0)

            one16 = jnp.full((16,), 1.0, jnp.float32)

            def orow(i, _):
                ones_v[i, pl.ds(0, 16)] = one16
                return 0
            lax.fori_loop(0, _CHUNK, orow, 0)

        pltpu.sync_copy(srcp.at[tid], src_v)
        pltpu.sync_copy(dstp.at[tid], dst_v)

        plsc.subcore_barrier()

        pltpu.async_copy(vals.at[src_v.at[0]], rb0, gsem)

        def chunk(j, _):
            even = j % 2 == 0

            def do(buf, nxt):
                @pl.when(j + 1 < _NCH)
                def _():
                    pltpu.async_copy(vals.at[src_v.at[j + 1]], nxt, gsem)
                pltpu.make_async_copy(vals.at[src_v.at[j]], buf, gsem).wait()
                pltpu.sync_copy(buf, acc.at[dst_v.at[j]], add=True)
                if with_deg:
                    pltpu.sync_copy(ones_v, accd.at[dst_v.at[j]], add=True)

            @pl.when(even)
            def _():
                do(rb0, rb1)

            @pl.when(jnp.logical_not(even))
            def _():
                do(rb1, rb0)
            return 0
        lax.fori_loop(0, _NCH, chunk, 0)

        plsc.subcore_barrier()

        ob = c * _NACC + rowb

        def wcp(k, _):
            pltpu.sync_copy(acc.at[pl.ds(rowb + k * _CHUNK, _CHUNK)], rb0)
            pltpu.sync_copy(rb0, out.at[pl.ds(ob + k * _CHUNK, _CHUNK)])
            return 0
        lax.fori_loop(0, _NZC, wcp, 0)

        if with_deg:
            def wdcp(k, _):
                pltpu.sync_copy(accd.at[pl.ds(rowb + k * _CHUNK, _CHUNK)], ones_v)
                pltpu.sync_copy(ones_v, outd.at[pl.ds(ob + k * _CHUNK, _CHUNK)])
                return 0
            lax.fori_loop(0, _NZC, wdcp, 0)

    return seg


def _tc_pre(x, W1l, W1r):
    def body(x_ref, a_ref, b_ref, y_ref, z_ref):
        xv = x_ref[...]
        y_ref[...] = jnp.dot(xv, a_ref[...], preferred_element_type=jnp.float32)
        z_ref[...] = jnp.dot(xv, b_ref[...], preferred_element_type=jnp.float32)

    return pl.pallas_call(
        body,
        out_shape=[jax.ShapeDtypeStruct((_N, _DH), jnp.float32),
                   jax.ShapeDtypeStruct((_N, _DH), jnp.float32)],
    )(x, W1l, W1r)


def _tc_mid(p1, pd, z1, b1, W2r):
    def body(p_ref, pd_ref, z1_ref, b1_ref, w_ref, h_ref, z2_ref):
        agg = p_ref[0:_N, :] + p_ref[_NACC:_NACC + _N, :]
        deg = pd_ref[0:_N, 0:1] + pd_ref[_NACC:_NACC + _N, 0:1]
        mean = agg / jnp.maximum(deg, 1.0)
        h = jnp.maximum(mean + b1_ref[...] + z1_ref[...], 0.0)
        h_ref[...] = h
        z2_ref[...] = jnp.dot(h, w_ref[...], preferred_element_type=jnp.float32)

    return pl.pallas_call(
        body,
        out_shape=[jax.ShapeDtypeStruct((_N, _DH), jnp.float32),
                   jax.ShapeDtypeStruct((_N, _DOUT), jnp.float32)],
    )(p1, pd, z1, b1, W2r)


def _tc_post(p2, pd, z2, b2, W2l):
    def body(p_ref, pd_ref, z2_ref, b2_ref, w_ref, o_ref):
        agg = p_ref[0:_N, :] + p_ref[_NACC:_NACC + _N, :]
        deg = pd_ref[0:_N, 0:1] + pd_ref[_NACC:_NACC + _N, 0:1]
        mean = agg / jnp.maximum(deg, 1.0)
        o_ref[...] = (jnp.dot(mean, w_ref[...], preferred_element_type=jnp.float32)
                      + b2_ref[...] + z2_ref[...])

    return pl.pallas_call(
        body,
        out_shape=jax.ShapeDtypeStruct((_N, _DOUT), jnp.float32),
    )(p2, pd, z2, b2, W2l)


def _first(res):
    return res[0] if isinstance(res, (list, tuple)) else res


def kernel(x, edge_index, W1l, b1, W1r, W2l, b2, W2r):
    src = edge_index[0]
    dst = edge_index[1]
    pad = _EPAD - _E
    ar = jnp.arange(pad, dtype=jnp.int32)
    srcp = jnp.concatenate([src, ar % 16]).reshape(_NT, _NCH, _CHUNK)
    dstp = jnp.concatenate([dst, _N + (ar % 16)]).reshape(_NT, _NCH, _CHUNK)

    y1, z1 = _tc_pre(x, W1l, W1r)
    p1, pd = _make_seg_sum(True)(y1, srcp, dstp)
    h, z2 = _tc_mid(p1, pd, z1, b1.reshape(1, _DH), W2r)
    p2 = _first(_make_seg_sum(False)(h, srcp, dstp))
    out = _tc_post(p2, pd, z2, b2.reshape(1, _DOUT), W2l)
    return out


# speedup vs baseline: 13.3221x; 1.0052x over previous
import functools

import jax
import jax.numpy as jnp
from jax import lax
from jax.experimental import pallas as pl
from jax.experimental.pallas import tpu as pltpu
from jax.experimental.pallas import tpu_sc as plsc

_N = 10000
_E = 320000
_DIN = 128
_DH = 64
_DOUT = 128

_NC = 2
_NS = 16
_NT = _NC * _NS
_PER_TILE = _E // _NT
_CH = 80
_NCHK = _PER_TILE // _CH
_NACC = 10240
_RPT = _NACC // _NS
_WB = 128
_NZC = _RPT // _WB
_DEGW = 16


@functools.lru_cache(maxsize=None)
def _make_seg_sum(with_deg):
    mesh = plsc.VectorSubcoreMesh(core_axis_name="c", subcore_axis_name="s")
    out_types = [jax.ShapeDtypeStruct((_NC * _NACC, _DH), jnp.float32)]
    scratch = [
        pltpu.VMEM((_NCHK, _CH), jnp.int32),
        pltpu.VMEM((_NCHK, _CH), jnp.int32),
        pltpu.VMEM((_CH, _DH), jnp.float32),
        pltpu.VMEM((_CH, _DH), jnp.float32),
        pltpu.VMEM((_WB, _DH), jnp.float32),
        pltpu.VMEM_SHARED((_NACC, _DH), jnp.float32),
        pltpu.SemaphoreType.DMA,
        pltpu.SemaphoreType.DMA,
    ]
    if with_deg:
        out_types.append(jax.ShapeDtypeStruct((_NC * _NACC, _DEGW), jnp.float32))
        scratch += [
            pltpu.VMEM((_CH, _DEGW), jnp.float32),
            pltpu.VMEM((_WB, _DEGW), jnp.float32),
            pltpu.VMEM_SHARED((_NACC, _DEGW), jnp.float32),
        ]

    @functools.partial(
        pl.kernel, mesh=mesh, out_type=out_types, scratch_types=scratch,
        compiler_params=pltpu.CompilerParams(use_tc_tiling_on_sc=False))
    def seg(vals, edge_r, *rest):
        if with_deg:
            (out, outd, src_v, dst_v, rb0, rb1, zbuf, acc, gsem, ssem,
             ones_v, zbufd, accd) = rest
        else:
            out, src_v, dst_v, rb0, rb1, zbuf, acc, gsem, ssem = rest
        c = lax.axis_index("c")
        s = lax.axis_index("s")
        tid = c * _NS + s
        rowb = s * _RPT

        zero16 = jnp.zeros((16,), jnp.float32)

        def zrow(i, _):
            for j in range(_DH // 16):
                zbuf[i, pl.ds(j * 16, 16)] = zero16
            return 0
        lax.fori_loop(0, _WB, zrow, 0)

        def zcp(k, _):
            pltpu.sync_copy(zbuf, acc.at[pl.ds(rowb + k * _WB, _WB)])
            return 0
        lax.fori_loop(0, _NZC, zcp, 0)

        if with_deg:
            def zdrow(i, _):
                zbufd[i, pl.ds(0, 16)] = zero16
                return 0
            lax.fori_loop(0, _WB, zdrow, 0)

            def zdcp(k, _):
                pltpu.sync_copy(zbufd, accd.at[pl.ds(rowb + k * _WB, _WB)])
                return 0
            lax.fori_loop(0, _NZC, zdcp, 0)

            one16 = jnp.full((16,), 1.0, jnp.float32)

            def orow(k, _):
                ones_v[k, pl.ds(0, 16)] = one16
                return 0
            lax.fori_loop(0, _CH, orow, 0)

        pltpu.sync_copy(edge_r.at[tid], src_v)
        pltpu.sync_copy(edge_r.at[_NT + tid], dst_v)

        plsc.subcore_barrier()

        def sidx(j):
            return j

        pltpu.async_copy(vals.at[src_v.at[sidx(0)]], rb0, gsem)

        def chunk(j, _):
            def do(buf, nxt):
                @pl.when(j + 1 < _NCHK)
                def _():
                    pltpu.async_copy(vals.at[src_v.at[sidx(j + 1)]], nxt, gsem)
                pltpu.make_async_copy(vals.at[src_v.at[sidx(j)]], buf,
                                      gsem).wait()
                d1 = pltpu.async_copy(buf, acc.at[dst_v.at[sidx(j)]], ssem,
                                      add=True)
                if with_deg:
                    d2 = pltpu.async_copy(ones_v, accd.at[dst_v.at[sidx(j)]],
                                          ssem, add=True)
                d1.wait()
                if with_deg:
                    d2.wait()

            @pl.when(j % 2 == 0)
            def _():
                do(rb0, rb1)

            @pl.when(j % 2 == 1)
            def _():
                do(rb1, rb0)
            return 0
        lax.fori_loop(0, _NCHK, chunk, 0)

        plsc.subcore_barrier()

        ob = c * _NACC + rowb

        def wcp(k, _):
            pltpu.sync_copy(acc.at[pl.ds(rowb + k * _WB, _WB)], zbuf)
            pltpu.sync_copy(zbuf, out.at[pl.ds(ob + k * _WB, _WB)])
            return 0
        lax.fori_loop(0, _NZC, wcp, 0)

        if with_deg:
            def wdcp(k, _):
                pltpu.sync_copy(accd.at[pl.ds(rowb + k * _WB, _WB)], zbufd)
                pltpu.sync_copy(zbufd, outd.at[pl.ds(ob + k * _WB, _WB)])
                return 0
            lax.fori_loop(0, _NZC, wdcp, 0)

    return seg


def _tc_pre(x, W1l, W1r):
    def body(x_ref, a_ref, b_ref, y_ref, z_ref):
        xv = x_ref[...]
        y_ref[...] = jnp.dot(xv, a_ref[...], preferred_element_type=jnp.float32)
        z_ref[...] = jnp.dot(xv, b_ref[...], preferred_element_type=jnp.float32)

    return pl.pallas_call(
        body,
        out_shape=[jax.ShapeDtypeStruct((_N, _DH), jnp.float32),
                   jax.ShapeDtypeStruct((_N, _DH), jnp.float32)],
    )(x, W1l, W1r)


def _tc_mid(p1, pd, z1, b1, W2r):
    def body(p_ref, pd_ref, z1_ref, b1_ref, w_ref, h_ref, z2_ref):
        agg = p_ref[0:_N, :] + p_ref[_NACC:_NACC + _N, :]
        deg = pd_ref[0:_N, 0:1] + pd_ref[_NACC:_NACC + _N, 0:1]
        mean = agg / jnp.maximum(deg, 1.0)
        h = jnp.maximum(mean + b1_ref[...] + z1_ref[...], 0.0)
        h_ref[...] = h
        z2_ref[...] = jnp.dot(h, w_ref[...], preferred_element_type=jnp.float32)

    return pl.pallas_call(
        body,
        out_shape=[jax.ShapeDtypeStruct((_N, _DH), jnp.float32),
                   jax.ShapeDtypeStruct((_N, _DOUT), jnp.float32)],
    )(p1, pd, z1, b1, W2r)


def _tc_post(p2, pd, z2, b2, W2l):
    def body(p_ref, pd_ref, z2_ref, b2_ref, w_ref, o_ref):
        agg = p_ref[0:_N, :] + p_ref[_NACC:_NACC + _N, :]
        deg = pd_ref[0:_N, 0:1] + pd_ref[_NACC:_NACC + _N, 0:1]
        mean = agg / jnp.maximum(deg, 1.0)
        o_ref[...] = (jnp.dot(mean, w_ref[...], preferred_element_type=jnp.float32)
                      + b2_ref[...] + z2_ref[...])

    return pl.pallas_call(
        body,
        out_shape=jax.ShapeDtypeStruct((_N, _DOUT), jnp.float32),
    )(p2, pd, z2, b2, W2l)


def _first(res):
    return res[0] if isinstance(res, (list, tuple)) else res


def kernel(x, edge_index, W1l, b1, W1r, W2l, b2, W2r):
    edge_r = edge_index.reshape(2 * _NT, _NCHK, _CH)

    y1, z1 = _tc_pre(x, W1l, W1r)
    p1, pd = _make_seg_sum(True)(y1, edge_r)
    h, z2 = _tc_mid(p1, pd, z1, b1.reshape(1, _DH), W2r)
    p2 = _first(_make_seg_sum(False)(h, edge_r))
    out = _tc_post(p2, pd, z2, b2.reshape(1, _DOUT), W2l)
    return out


# speedup vs baseline: 14.3029x; 1.0736x over previous
import functools

import jax
import jax.numpy as jnp
from jax import lax
from jax.experimental import pallas as pl
from jax.experimental.pallas import tpu as pltpu
from jax.experimental.pallas import tpu_sc as plsc

_N = 10000
_E = 320000
_DIN = 128
_DH = 64
_DOUT = 128
_W1 = _DH + 16

_NC = 2
_NS = 16
_NT = _NC * _NS
_PER_TILE = _E // _NT
_CH = 80
_NCHK = _PER_TILE // _CH
_NACC = 10240
_RPT = _NACC // _NS
_WB = 128
_NZC = _RPT // _WB
_NBUF = 4


@functools.lru_cache(maxsize=None)
def _make_seg_sum(width):
    mesh = plsc.VectorSubcoreMesh(core_axis_name="c", subcore_axis_name="s")
    scratch = (
        [pltpu.VMEM((_NCHK, _CH), jnp.int32)] * 2 +
        [pltpu.VMEM((_CH, width), jnp.float32)] * _NBUF +
        [pltpu.VMEM((_WB, width), jnp.float32)] +
        [pltpu.VMEM_SHARED((_NACC, width), jnp.float32)] +
        [pltpu.SemaphoreType.DMA] * (2 * _NBUF)
    )

    @functools.partial(
        pl.kernel, mesh=mesh,
        out_type=jax.ShapeDtypeStruct((_NC * _NACC, width), jnp.float32),
        scratch_types=scratch,
        compiler_params=pltpu.CompilerParams(use_tc_tiling_on_sc=False))
    def seg(vals, edge_r, out, src_v, dst_v, *rest):
        rbufs = rest[:_NBUF]
        zbuf = rest[_NBUF]
        acc = rest[_NBUF + 1]
        gsems = rest[_NBUF + 2:_NBUF + 2 + _NBUF]
        ssems = rest[_NBUF + 2 + _NBUF:]
        c = lax.axis_index("c")
        s = lax.axis_index("s")
        tid = c * _NS + s
        rowb = s * _RPT

        zero16 = jnp.zeros((16,), jnp.float32)

        def zrow(i, _):
            for j in range(width // 16):
                zbuf[i, pl.ds(j * 16, 16)] = zero16
            return 0
        lax.fori_loop(0, _WB, zrow, 0)

        def zcp(k, _):
            pltpu.sync_copy(zbuf, acc.at[pl.ds(rowb + k * _WB, _WB)])
            return 0
        lax.fori_loop(0, _NZC, zcp, 0)

        pltpu.sync_copy(edge_r.at[tid], src_v)
        pltpu.sync_copy(edge_r.at[_NT + tid], dst_v)

        plsc.subcore_barrier()

        pltpu.async_copy(vals.at[src_v.at[0]], rbufs[0], gsems[0])
        pltpu.async_copy(vals.at[src_v.at[1]], rbufs[1], gsems[1])

        def chunk(j, _):
            for b in range(_NBUF):
                @pl.when(j % _NBUF == b)
                def _(b=b):
                    nb = (b + 2) % _NBUF
                    pltpu.make_async_copy(vals.at[src_v.at[j]], rbufs[b],
                                          gsems[b]).wait()
                    pltpu.async_copy(rbufs[b], acc.at[dst_v.at[j]],
                                     ssems[b], add=True)

                    @pl.when(j >= 2)
                    def _():
                        pltpu.make_async_copy(rbufs[nb], acc.at[dst_v.at[j]],
                                              ssems[nb]).wait()

                    @pl.when(j + 2 < _NCHK)
                    def _():
                        pltpu.async_copy(vals.at[src_v.at[j + 2]],
                                         rbufs[nb], gsems[nb])
            return 0
        lax.fori_loop(0, _NCHK, chunk, 0)

        for jj in (_NCHK - 2, _NCHK - 1):
            pltpu.make_async_copy(rbufs[jj % _NBUF], acc.at[dst_v.at[0]],
                                  ssems[jj % _NBUF]).wait()

        plsc.subcore_barrier()

        ob = c * _NACC + rowb

        def wcp(k, _):
            pltpu.sync_copy(acc.at[pl.ds(rowb + k * _WB, _WB)], zbuf)
            pltpu.sync_copy(zbuf, out.at[pl.ds(ob + k * _WB, _WB)])
            return 0
        lax.fori_loop(0, _NZC, wcp, 0)

    return seg


def _tc_pre(x, W1l, W1r):
    def body(x_ref, a_ref, b_ref, y_ref, z_ref):
        xv = x_ref[...]
        y_ref[:, 0:_DH] = jnp.dot(xv, a_ref[...],
                                  preferred_element_type=jnp.float32)
        y_ref[:, _DH:_W1] = jnp.ones((_N, _W1 - _DH), jnp.float32)
        z_ref[...] = jnp.dot(xv, b_ref[...], preferred_element_type=jnp.float32)

    return pl.pallas_call(
        body,
        out_shape=[jax.ShapeDtypeStruct((_N, _W1), jnp.float32),
                   jax.ShapeDtypeStruct((_N, _DH), jnp.float32)],
    )(x, W1l, W1r)


def _tc_mid(p1, z1, b1, W2r):
    def body(p_ref, z1_ref, b1_ref, w_ref, h_ref, z2_ref, d_ref):
        agg = p_ref[0:_N, 0:_DH] + p_ref[_NACC:_NACC + _N, 0:_DH]
        deg = p_ref[0:_N, _DH:_DH + 1] + p_ref[_NACC:_NACC + _N, _DH:_DH + 1]
        degc = jnp.maximum(deg, 1.0)
        mean = agg / degc
        h = jnp.maximum(mean + b1_ref[...] + z1_ref[...], 0.0)
        h_ref[...] = h
        z2_ref[...] = jnp.dot(h, w_ref[...], preferred_element_type=jnp.float32)
        d_ref[...] = jnp.broadcast_to(degc, (_N, 8))

    return pl.pallas_call(
        body,
        out_shape=[jax.ShapeDtypeStruct((_N, _DH), jnp.float32),
                   jax.ShapeDtypeStruct((_N, _DOUT), jnp.float32),
                   jax.ShapeDtypeStruct((_N, 8), jnp.float32)],
    )(p1, z1, b1, W2r)


def _tc_post(p2, degc, z2, b2, W2l):
    def body(p_ref, d_ref, z2_ref, b2_ref, w_ref, o_ref):
        agg = p_ref[0:_N, :] + p_ref[_NACC:_NACC + _N, :]
        mean = agg / d_ref[:, 0:1]
        o_ref[...] = (jnp.dot(mean, w_ref[...], preferred_element_type=jnp.float32)
                      + b2_ref[...] + z2_ref[...])

    return pl.pallas_call(
        body,
        out_shape=jax.ShapeDtypeStruct((_N, _DOUT), jnp.float32),
    )(p2, degc, z2, b2, W2l)


def _first(res):
    return res[0] if isinstance(res, (list, tuple)) else res


def kernel(x, edge_index, W1l, b1, W1r, W2l, b2, W2r):
    edge_r = edge_index.reshape(2 * _NT, _NCHK, _CH)

    y1p, z1 = _tc_pre(x, W1l, W1r)
    p1 = _first(_make_seg_sum(_W1)(y1p, edge_r))
    h, z2, degc = _tc_mid(p1, z1, b1.reshape(1, _DH), W2r)
    p2 = _first(_make_seg_sum(_DH)(h, edge_r))
    out = _tc_post(p2, degc, z2, b2.reshape(1, _DOUT), W2l)
    return out


# speedup vs baseline: 15.5279x; 1.0856x over previous
import functools

import jax
import jax.numpy as jnp
from jax import lax
from jax.experimental import pallas as pl
from jax.experimental.pallas import tpu as pltpu
from jax.experimental.pallas import tpu_sc as plsc

_N = 10000
_E = 320000
_DIN = 128
_DH = 64
_DOUT = 128

_NC = 2
_NS = 16
_NT = _NC * _NS
_CH = 128
_NCHK = 79
_PER_TILE = _CH * _NCHK
_EPAD = _NT * _PER_TILE
_NACC = 10240
_RPT = _NACC // _NS
_SRT = 128
_DEGW = 16
_NBUF = 4
_GD = 3
_SL = _NBUF - _GD


@functools.lru_cache(maxsize=None)
def _make_seg_sum(with_deg):
    mesh = plsc.VectorSubcoreMesh(core_axis_name="c", subcore_axis_name="s")
    out_types = [jax.ShapeDtypeStruct((_NC * _NACC, _DH), jnp.float32)]
    scratch = (
        [pltpu.VMEM((_NCHK, _CH), jnp.int32)] * 2 +
        [pltpu.VMEM((_CH, _DH), jnp.float32)] * _NBUF +
        [pltpu.VMEM((_SRT, _DH), jnp.float32)] +
        [pltpu.VMEM_SHARED((_NACC, _DH), jnp.float32)] +
        [pltpu.SemaphoreType.DMA] * (2 * _NBUF)
    )
    if with_deg:
        out_types.append(jax.ShapeDtypeStruct((_NC * _NACC, _DEGW),
                                              jnp.float32))
        scratch += [
            pltpu.VMEM((_CH, _DEGW), jnp.float32),
            pltpu.VMEM((_SRT, _DEGW), jnp.float32),
            pltpu.VMEM_SHARED((_NACC, _DEGW), jnp.float32),
        ]

    @functools.partial(
        pl.kernel, mesh=mesh, out_type=out_types, scratch_types=scratch,
        compiler_params=pltpu.CompilerParams(use_tc_tiling_on_sc=False))
    def seg(vals, edge_r, *rest):
        if with_deg:
            (out, outd, src_v, dst_v, r0, r1, r2, r3, zbuf, acc,
             g0, g1, g2, g3, s0, s1, s2, s3,
             ones_v, zbufd, accd) = rest
        else:
            (out, src_v, dst_v, r0, r1, r2, r3, zbuf, acc,
             g0, g1, g2, g3, s0, s1, s2, s3) = rest
        rbufs = (r0, r1, r2, r3)
        gsems = (g0, g1, g2, g3)
        ssems = (s0, s1, s2, s3)
        c = lax.axis_index("c")
        s = lax.axis_index("s")
        tid = c * _NS + s
        rowb = s * _RPT

        zero16 = jnp.zeros((16,), jnp.float32)

        def zrow(i, _):
            for j in range(_DH // 16):
                zbuf[i, pl.ds(j * 16, 16)] = zero16
            return 0
        lax.fori_loop(0, _SRT, zrow, 0)
        for k in range(_RPT // _SRT):
            pltpu.sync_copy(zbuf, acc.at[pl.ds(rowb + k * _SRT, _SRT)])

        if with_deg:
            def zdrow(i, _):
                zbufd[i, pl.ds(0, 16)] = zero16
                return 0
            lax.fori_loop(0, _SRT, zdrow, 0)
            for k in range(_RPT // _SRT):
                pltpu.sync_copy(zbufd, accd.at[pl.ds(rowb + k * _SRT, _SRT)])

            one16 = jnp.full((16,), 1.0, jnp.float32)

            def orow(i, _):
                ones_v[i, pl.ds(0, 16)] = one16
                return 0
            lax.fori_loop(0, _CH, orow, 0)

        pltpu.sync_copy(edge_r.at[tid], src_v)
        pltpu.sync_copy(edge_r.at[_NT + tid], dst_v)

        plsc.subcore_barrier()

        for g in range(_GD):
            pltpu.async_copy(vals.at[src_v.at[g]], rbufs[g], gsems[g])

        def chunk(j, _):
            for b in range(_NBUF):
                @pl.when(j % _NBUF == b)
                def _(b=b):
                    nb = (b + _GD) % _NBUF
                    pltpu.make_async_copy(vals.at[src_v.at[j]], rbufs[b],
                                          gsems[b]).wait()
                    pltpu.async_copy(rbufs[b], acc.at[dst_v.at[j]],
                                     ssems[b], add=True)
                    if with_deg:
                        pltpu.async_copy(ones_v, accd.at[dst_v.at[j]],
                                         ssems[b], add=True)

                    @pl.when(j >= _SL)
                    def _():
                        pltpu.make_async_copy(rbufs[nb], acc.at[dst_v.at[j]],
                                              ssems[nb]).wait()
                        if with_deg:
                            pltpu.make_async_copy(
                                ones_v, accd.at[dst_v.at[j]],
                                ssems[nb]).wait()

                    @pl.when(j + _GD < _NCHK)
                    def _():
                        pltpu.async_copy(vals.at[src_v.at[j + _GD]],
                                         rbufs[nb], gsems[nb])
            return 0
        lax.fori_loop(0, _NCHK, chunk, 0)

        for jj in range(_NCHK - _SL, _NCHK):
            pltpu.make_async_copy(rbufs[jj % _NBUF], acc.at[dst_v.at[0]],
                                  ssems[jj % _NBUF]).wait()
            if with_deg:
                pltpu.make_async_copy(ones_v, accd.at[dst_v.at[0]],
                                      ssems[jj % _NBUF]).wait()

        plsc.subcore_barrier()

        ob = c * _NACC + rowb
        for k in range(_RPT // _SRT):
            pltpu.sync_copy(acc.at[pl.ds(rowb + k * _SRT, _SRT)], zbuf)
            pltpu.sync_copy(zbuf, out.at[pl.ds(ob + k * _SRT, _SRT)])
        if with_deg:
            for k in range(_RPT // _SRT):
                pltpu.sync_copy(accd.at[pl.ds(rowb + k * _SRT, _SRT)], zbufd)
                pltpu.sync_copy(zbufd, outd.at[pl.ds(ob + k * _SRT, _SRT)])

    return seg


def _tc_pre(x, W1l, W1r):
    def body(x_ref, a_ref, b_ref, y_ref, z_ref):
        xv = x_ref[...]
        y_ref[...] = jnp.dot(xv, a_ref[...], preferred_element_type=jnp.float32)
        z_ref[...] = jnp.dot(xv, b_ref[...], preferred_element_type=jnp.float32)

    return pl.pallas_call(
        body,
        out_shape=[jax.ShapeDtypeStruct((_N, _DH), jnp.float32),
                   jax.ShapeDtypeStruct((_N, _DH), jnp.float32)],
    )(x, W1l, W1r)


def _tc_mid(p1, pd, z1, b1, W2r):
    def body(p_ref, pd_ref, z1_ref, b1_ref, w_ref, h_ref, z2_ref, d_ref):
        agg = p_ref[0:_N, :] + p_ref[_NACC:_NACC + _N, :]
        deg = pd_ref[0:_N, 0:1] + pd_ref[_NACC:_NACC + _N, 0:1]
        degc = jnp.maximum(deg, 1.0)
        mean = agg / degc
        h = jnp.maximum(mean + b1_ref[...] + z1_ref[...], 0.0)
        h_ref[...] = h
        z2_ref[...] = jnp.dot(h, w_ref[...], preferred_element_type=jnp.float32)
        d_ref[...] = jnp.broadcast_to(degc, (_N, 8))

    return pl.pallas_call(
        body,
        out_shape=[jax.ShapeDtypeStruct((_N, _DH), jnp.float32),
                   jax.ShapeDtypeStruct((_N, _DOUT), jnp.float32),
                   jax.ShapeDtypeStruct((_N, 8), jnp.float32)],
    )(p1, pd, z1, b1, W2r)


def _tc_post(p2, degc, z2, b2, W2l):
    def body(p_ref, d_ref, z2_ref, b2_ref, w_ref, o_ref):
        agg = p_ref[0:_N, :] + p_ref[_NACC:_NACC + _N, :]
        mean = agg / d_ref[:, 0:1]
        o_ref[...] = (jnp.dot(mean, w_ref[...], preferred_element_type=jnp.float32)
                      + b2_ref[...] + z2_ref[...])

    return pl.pallas_call(
        body,
        out_shape=jax.ShapeDtypeStruct((_N, _DOUT), jnp.float32),
    )(p2, degc, z2, b2, W2l)


def _first(res):
    return res[0] if isinstance(res, (list, tuple)) else res


def kernel(x, edge_index, W1l, b1, W1r, W2l, b2, W2r):
    ar = jnp.arange(_EPAD - _E, dtype=jnp.int32) % 16
    padblk = jnp.stack([ar, _N + ar])
    edge_r = jnp.concatenate([edge_index, padblk], axis=1).reshape(
        2 * _NT, _NCHK, _CH)

    y1, z1 = _tc_pre(x, W1l, W1r)
    p1, pd = _make_seg_sum(True)(y1, edge_r)
    h, z2, degc = _tc_mid(p1, pd, z1, b1.reshape(1, _DH), W2r)
    p2 = _first(_make_seg_sum(False)(h, edge_r))
    out = _tc_post(p2, degc, z2, b2.reshape(1, _DOUT), W2l)
    return out


# speedup vs baseline: 15.8371x; 1.0199x over previous
import functools

import jax
import jax.numpy as jnp
from jax import lax
from jax.experimental import pallas as pl
from jax.experimental.pallas import tpu as pltpu
from jax.experimental.pallas import tpu_sc as plsc

_N = 10000
_E = 320000
_DIN = 128
_DH = 64
_DOUT = 128

_NC = 2
_NS = 16
_NT = _NC * _NS
_CH = 128
_NCHK = 79
_PER_TILE = _CH * _NCHK
_EPAD = _NT * _PER_TILE
_NACC = 10240
_RPT = _NACC // _NS
_SRT = 128
_DEGW = 16
_NBUF = 4
_GD = 3
_SL = _NBUF - _GD


@functools.lru_cache(maxsize=None)
def _make_seg_sum(with_deg):
    mesh = plsc.VectorSubcoreMesh(core_axis_name="c", subcore_axis_name="s")
    out_types = [jax.ShapeDtypeStruct((_NC * _NACC, _DH), jnp.float32)]
    scratch = (
        [pltpu.VMEM((_NCHK, _CH), jnp.int32)] * 2 +
        [pltpu.VMEM((_CH, _DH), jnp.float32)] * _NBUF +
        [pltpu.VMEM((_SRT, _DH), jnp.float32)] +
        [pltpu.VMEM_SHARED((_NACC, _DH), jnp.float32)] +
        [pltpu.SemaphoreType.DMA] * (2 * _NBUF)
    )
    if with_deg:
        out_types.append(jax.ShapeDtypeStruct((_NC * _NACC, _DEGW),
                                              jnp.float32))
        scratch += [
            pltpu.VMEM((_CH, _DEGW), jnp.float32),
            pltpu.VMEM((_SRT, _DEGW), jnp.float32),
            pltpu.VMEM_SHARED((_NACC, _DEGW), jnp.float32),
        ]

    @functools.partial(
        pl.kernel, mesh=mesh, out_type=out_types, scratch_types=scratch,
        compiler_params=pltpu.CompilerParams(use_tc_tiling_on_sc=False))
    def seg(vals, edge_r, *rest):
        if with_deg:
            (out, outd, src_v, dst_v, r0, r1, r2, r3, zbuf, acc,
             g0, g1, g2, g3, s0, s1, s2, s3,
             ones_v, zbufd, accd) = rest
        else:
            (out, src_v, dst_v, r0, r1, r2, r3, zbuf, acc,
             g0, g1, g2, g3, s0, s1, s2, s3) = rest
        rbufs = (r0, r1, r2, r3)
        gsems = (g0, g1, g2, g3)
        ssems = (s0, s1, s2, s3)
        c = lax.axis_index("c")
        s = lax.axis_index("s")
        tid = c * _NS + s
        rowb = s * _RPT

        zero16 = jnp.zeros((16,), jnp.float32)

        def zrow(i, _):
            for j in range(_DH // 16):
                zbuf[i, pl.ds(j * 16, 16)] = zero16
            return 0
        lax.fori_loop(0, _SRT, zrow, 0)
        for k in range(_RPT // _SRT):
            pltpu.sync_copy(zbuf, acc.at[pl.ds(rowb + k * _SRT, _SRT)])

        if with_deg:
            def zdrow(i, _):
                zbufd[i, pl.ds(0, 16)] = zero16
                return 0
            lax.fori_loop(0, _SRT, zdrow, 0)
            for k in range(_RPT // _SRT):
                pltpu.sync_copy(zbufd, accd.at[pl.ds(rowb + k * _SRT, _SRT)])

            one16 = jnp.full((16,), 1.0, jnp.float32)

            def orow(i, _):
                ones_v[i, pl.ds(0, 16)] = one16
                return 0
            lax.fori_loop(0, _CH, orow, 0)

        pltpu.sync_copy(edge_r.at[tid], src_v)
        pltpu.sync_copy(edge_r.at[_NT + tid], dst_v)

        plsc.subcore_barrier()

        for g in range(_GD):
            pltpu.async_copy(vals.at[src_v.at[g]], rbufs[g], gsems[g])

        def chunk(j, _):
            for b in range(_NBUF):
                @pl.when(j % _NBUF == b)
                def _(b=b):
                    nb = (b + _GD) % _NBUF
                    pltpu.make_async_copy(vals.at[src_v.at[j]], rbufs[b],
                                          gsems[b]).wait()
                    pltpu.async_copy(rbufs[b], acc.at[dst_v.at[j]],
                                     ssems[b], add=True)
                    if with_deg:
                        pltpu.async_copy(ones_v, accd.at[dst_v.at[j]],
                                         ssems[b], add=True)

                    @pl.when(j >= _SL)
                    def _():
                        pltpu.make_async_copy(rbufs[nb], acc.at[dst_v.at[j]],
                                              ssems[nb]).wait()
                        if with_deg:
                            pltpu.make_async_copy(
                                ones_v, accd.at[dst_v.at[j]],
                                ssems[nb]).wait()

                    @pl.when(j + _GD < _NCHK)
                    def _():
                        pltpu.async_copy(vals.at[src_v.at[j + _GD]],
                                         rbufs[nb], gsems[nb])
            return 0
        lax.fori_loop(0, _NCHK, chunk, 0)

        for jj in range(_NCHK - _SL, _NCHK):
            pltpu.make_async_copy(rbufs[jj % _NBUF], acc.at[dst_v.at[0]],
                                  ssems[jj % _NBUF]).wait()
            if with_deg:
                pltpu.make_async_copy(ones_v, accd.at[dst_v.at[0]],
                                      ssems[jj % _NBUF]).wait()

        plsc.subcore_barrier()

        ob = c * _NACC + rowb
        for k in range(_RPT // _SRT):
            pltpu.sync_copy(acc.at[pl.ds(rowb + k * _SRT, _SRT)], zbuf)
            pltpu.sync_copy(zbuf, out.at[pl.ds(ob + k * _SRT, _SRT)])
        if with_deg:
            for k in range(_RPT // _SRT):
                pltpu.sync_copy(accd.at[pl.ds(rowb + k * _SRT, _SRT)], zbufd)
                pltpu.sync_copy(zbufd, outd.at[pl.ds(ob + k * _SRT, _SRT)])

    return seg


def _tc_proj(x, W):
    def body(x_ref, w_ref, y_ref):
        y_ref[...] = jnp.dot(x_ref[...], w_ref[...],
                             preferred_element_type=jnp.float32)

    return pl.pallas_call(
        body,
        out_shape=jax.ShapeDtypeStruct((x.shape[0], W.shape[1]), jnp.float32),
    )(x, W)


def _tc_mid(p1, pd, z1, b1):
    def body(p_ref, pd_ref, z1_ref, b1_ref, h_ref, d_ref):
        agg = p_ref[0:_N, :] + p_ref[_NACC:_NACC + _N, :]
        deg = pd_ref[0:_N, 0:1] + pd_ref[_NACC:_NACC + _N, 0:1]
        degc = jnp.maximum(deg, 1.0)
        mean = agg / degc
        h = jnp.maximum(mean + b1_ref[...] + z1_ref[...], 0.0)
        h_ref[...] = h
        d_ref[...] = jnp.broadcast_to(degc, (_N, 8))

    return pl.pallas_call(
        body,
        out_shape=[jax.ShapeDtypeStruct((_N, _DH), jnp.float32),
                   jax.ShapeDtypeStruct((_N, 8), jnp.float32)],
    )(p1, pd, z1, b1)


def _tc_post(p2, degc, z2, b2, W2l):
    def body(p_ref, d_ref, z2_ref, b2_ref, w_ref, o_ref):
        agg = p_ref[0:_N, :] + p_ref[_NACC:_NACC + _N, :]
        mean = agg / d_ref[:, 0:1]
        o_ref[...] = (jnp.dot(mean, w_ref[...], preferred_element_type=jnp.float32)
                      + b2_ref[...] + z2_ref[...])

    return pl.pallas_call(
        body,
        out_shape=jax.ShapeDtypeStruct((_N, _DOUT), jnp.float32),
    )(p2, degc, z2, b2, W2l)


def _first(res):
    return res[0] if isinstance(res, (list, tuple)) else res


def kernel(x, edge_index, W1l, b1, W1r, W2l, b2, W2r):
    ar = jnp.arange(_EPAD - _E, dtype=jnp.int32) % 16
    padblk = jnp.stack([ar, _N + ar])
    edge_r = jnp.concatenate([edge_index, padblk], axis=1).reshape(
        2 * _NT, _NCHK, _CH)

    y1 = _tc_proj(x, W1l)
    p1, pd = _make_seg_sum(True)(y1, edge_r)
    z1 = _tc_proj(x, W1r)
    h, degc = _tc_mid(p1, pd, z1, b1.reshape(1, _DH))
    p2 = _first(_make_seg_sum(False)(h, edge_r))
    z2 = _tc_proj(h, W2r)
    out = _tc_post(p2, degc, z2, b2.reshape(1, _DOUT), W2l)
    return out


# speedup vs baseline: 16.3399x; 1.0317x over previous
import functools

import jax
import jax.numpy as jnp
from jax import lax
from jax.experimental import pallas as pl
from jax.experimental.pallas import tpu as pltpu
from jax.experimental.pallas import tpu_sc as plsc

_N = 10000
_E = 320000
_DIN = 128
_DH = 64
_DOUT = 128

_NC = 2
_NS = 16
_NT = _NC * _NS
_CH = 128
_NCHK = 79
_PER_TILE = _CH * _NCHK
_EPAD = _NT * _PER_TILE
_NACC = 10240
_RPT = _NACC // _NS
_SRT = 128
_DEGW = 16
_NBUF = 4
_GD = 3
_SL = _NBUF - _GD


@functools.lru_cache(maxsize=None)
def _make_seg_sum(with_deg):
    mesh = plsc.VectorSubcoreMesh(core_axis_name="c", subcore_axis_name="s")
    out_types = [jax.ShapeDtypeStruct((_NC * _NACC, _DH), jnp.float32)]
    scratch = (
        [pltpu.VMEM((_NCHK, _CH), jnp.int32)] * 2 +
        [pltpu.VMEM((_CH, _DH), jnp.float32)] * _NBUF +
        [pltpu.VMEM((_SRT, _DH), jnp.float32)] +
        [pltpu.VMEM_SHARED((_NACC, _DH), jnp.float32)] +
        [pltpu.SemaphoreType.DMA] * (2 * _NBUF)
    )
    if with_deg:
        out_types.append(jax.ShapeDtypeStruct((_NC * _NACC, _DEGW),
                                              jnp.float32))
        scratch += [
            pltpu.VMEM((_CH, _DEGW), jnp.float32),
            pltpu.VMEM((_SRT, _DEGW), jnp.float32),
            pltpu.VMEM_SHARED((_NACC, _DEGW), jnp.float32),
        ]

    @functools.partial(
        pl.kernel, mesh=mesh, out_type=out_types, scratch_types=scratch,
        compiler_params=pltpu.CompilerParams(use_tc_tiling_on_sc=False))
    def seg(vals, edge_r, *rest):
        if with_deg:
            (out, outd, src_v, dst_v, r0, r1, r2, r3, zbuf, acc,
             g0, g1, g2, g3, s0, s1, s2, s3,
             ones_v, zbufd, accd) = rest
        else:
            (out, src_v, dst_v, r0, r1, r2, r3, zbuf, acc,
             g0, g1, g2, g3, s0, s1, s2, s3) = rest
        rbufs = (r0, r1, r2, r3)
        gsems = (g0, g1, g2, g3)
        ssems = (s0, s1, s2, s3)
        c = lax.axis_index("c")
        s = lax.axis_index("s")
        tid = c * _NS + s
        rowb = s * _RPT

        zero16 = jnp.zeros((16,), jnp.float32)

        pltpu.async_copy(edge_r.at[tid], src_v, gsems[0])
        pltpu.async_copy(edge_r.at[_NT + tid], dst_v, gsems[1])

        def zrow(i, _):
            for j in range(_DH // 16):
                zbuf[i, pl.ds(j * 16, 16)] = zero16
            return 0
        lax.fori_loop(0, _SRT, zrow, 0)
        for k in range(_RPT // _SRT):
            pltpu.async_copy(zbuf, acc.at[pl.ds(rowb + k * _SRT, _SRT)],
                             ssems[0])

        if with_deg:
            def zdrow(i, _):
                zbufd[i, pl.ds(0, 16)] = zero16
                return 0
            lax.fori_loop(0, _SRT, zdrow, 0)
            for k in range(_RPT // _SRT):
                pltpu.async_copy(zbufd, accd.at[pl.ds(rowb + k * _SRT, _SRT)],
                                 ssems[1])

            one16 = jnp.full((16,), 1.0, jnp.float32)

            def orow(i, _):
                ones_v[i, pl.ds(0, 16)] = one16
                return 0
            lax.fori_loop(0, _CH, orow, 0)

        for k in range(_RPT // _SRT):
            pltpu.make_async_copy(zbuf, acc.at[pl.ds(rowb, _SRT)],
                                  ssems[0]).wait()
            if with_deg:
                pltpu.make_async_copy(zbufd, accd.at[pl.ds(rowb, _SRT)],
                                      ssems[1]).wait()
        pltpu.make_async_copy(edge_r.at[tid], src_v, gsems[0]).wait()
        pltpu.make_async_copy(edge_r.at[_NT + tid], dst_v, gsems[1]).wait()

        plsc.subcore_barrier()

        for g in range(_GD):
            pltpu.async_copy(vals.at[src_v.at[g]], rbufs[g], gsems[g])

        def chunk(j, _):
            for b in range(_NBUF):
                @pl.when(j % _NBUF == b)
                def _(b=b):
                    nb = (b + _GD) % _NBUF
                    pltpu.make_async_copy(vals.at[src_v.at[j]], rbufs[b],
                                          gsems[b]).wait()
                    pltpu.async_copy(rbufs[b], acc.at[dst_v.at[j]],
                                     ssems[b], add=True)
                    if with_deg:
                        pltpu.async_copy(ones_v, accd.at[dst_v.at[j]],
                                         ssems[b], add=True)

                    @pl.when(j >= _SL)
                    def _():
                        pltpu.make_async_copy(rbufs[nb], acc.at[dst_v.at[j]],
                                              ssems[nb]).wait()
                        if with_deg:
                            pltpu.make_async_copy(
                                ones_v, accd.at[dst_v.at[j]],
                                ssems[nb]).wait()

                    @pl.when(j + _GD < _NCHK)
                    def _():
                        pltpu.async_copy(vals.at[src_v.at[j + _GD]],
                                         rbufs[nb], gsems[nb])
            return 0
        lax.fori_loop(0, _NCHK, chunk, 0)

        for jj in range(_NCHK - _SL, _NCHK):
            pltpu.make_async_copy(rbufs[jj % _NBUF], acc.at[dst_v.at[0]],
                                  ssems[jj % _NBUF]).wait()
            if with_deg:
                pltpu.make_async_copy(ones_v, accd.at[dst_v.at[0]],
                                      ssems[jj % _NBUF]).wait()

        plsc.subcore_barrier()

        ob = c * _NACC + rowb
        pltpu.async_copy(acc.at[pl.ds(rowb, _RPT)],
                         out.at[pl.ds(ob, _RPT)], gsems[0])
        if with_deg:
            pltpu.async_copy(accd.at[pl.ds(rowb, _RPT)],
                             outd.at[pl.ds(ob, _RPT)], gsems[1])
            pltpu.make_async_copy(accd.at[pl.ds(rowb, _RPT)],
                                  outd.at[pl.ds(ob, _RPT)], gsems[1]).wait()
        pltpu.make_async_copy(acc.at[pl.ds(rowb, _RPT)],
                              out.at[pl.ds(ob, _RPT)], gsems[0]).wait()

    return seg


def _tc_proj(x, W):
    def body(x_ref, w_ref, y_ref):
        y_ref[...] = jnp.dot(x_ref[...], w_ref[...],
                             preferred_element_type=jnp.float32)

    return pl.pallas_call(
        body,
        out_shape=jax.ShapeDtypeStruct((x.shape[0], W.shape[1]), jnp.float32),
    )(x, W)


def _tc_mid(p1, pd, z1, b1):
    def body(p_ref, pd_ref, z1_ref, b1_ref, h_ref, d_ref):
        agg = p_ref[0:_N, :] + p_ref[_NACC:_NACC + _N, :]
        deg = pd_ref[0:_N, 0:1] + pd_ref[_NACC:_NACC + _N, 0:1]
        degc = jnp.maximum(deg, 1.0)
        mean = agg / degc
        h = jnp.maximum(mean + b1_ref[...] + z1_ref[...], 0.0)
        h_ref[...] = h
        d_ref[...] = jnp.broadcast_to(degc, (_N, 8))

    return pl.pallas_call(
        body,
        out_shape=[jax.ShapeDtypeStruct((_N, _DH), jnp.float32),
                   jax.ShapeDtypeStruct((_N, 8), jnp.float32)],
    )(p1, pd, z1, b1)


def _tc_post(p2, degc, z2, b2, W2l):
    def body(p_ref, d_ref, z2_ref, b2_ref, w_ref, o_ref):
        agg = p_ref[0:_N, :] + p_ref[_NACC:_NACC + _N, :]
        mean = agg / d_ref[:, 0:1]
        o_ref[...] = (jnp.dot(mean, w_ref[...], preferred_element_type=jnp.float32)
                      + b2_ref[...] + z2_ref[...])

    return pl.pallas_call(
        body,
        out_shape=jax.ShapeDtypeStruct((_N, _DOUT), jnp.float32),
    )(p2, degc, z2, b2, W2l)


def _first(res):
    return res[0] if isinstance(res, (list, tuple)) else res


def kernel(x, edge_index, W1l, b1, W1r, W2l, b2, W2r):
    ar = jnp.arange(_EPAD - _E, dtype=jnp.int32) % 16
    padblk = jnp.stack([ar, _N + ar])
    edge_r = jnp.concatenate([edge_index, padblk], axis=1).reshape(
        2 * _NT, _NCHK, _CH)

    y1 = _tc_proj(x, W1l)
    p1, pd = _make_seg_sum(True)(y1, edge_r)
    z1 = _tc_proj(x, W1r)
    h, degc = _tc_mid(p1, pd, z1, b1.reshape(1, _DH))
    p2 = _first(_make_seg_sum(False)(h, edge_r))
    z2 = _tc_proj(h, W2r)
    out = _tc_post(p2, degc, z2, b2.reshape(1, _DOUT), W2l)
    return out
